# baseline (speedup 1.0000x reference)
import functools

import jax
import jax.numpy as jnp
from jax import lax
from jax.experimental import pallas as pl
from jax.experimental.pallas import tpu as pltpu
from jax.experimental.pallas import tpu_sc as plsc

_NC = 2
_NS = 16
_NW = _NC * _NS

_IPR = 512
_NB = 4
_UNROLL = 8


@functools.lru_cache(maxsize=None)
def _build(n_rows: int, dim: int):
    odim = _IPR * dim
    spans = _IPR // 4
    assert n_rows % (_NW * _NB) == 0 and spans % _UNROLL == 0
    r_per_w = n_rows // _NW
    n_groups = r_per_w // _NB
    mesh = plsc.VectorSubcoreMesh(core_axis_name="c", subcore_axis_name="s")

    @functools.partial(
        pl.kernel,
        mesh=mesh,
        out_type=jax.ShapeDtypeStruct((n_rows * odim,), jnp.float32),
        scratch_types=[
            pltpu.VMEM((9, dim), jnp.float32),
            [pltpu.VMEM((_IPR,), jnp.int32)] * _NB,
            [pltpu.VMEM((odim,), jnp.float32)] * _NB,
            pltpu.SemaphoreType.DMA,
            pltpu.SemaphoreType.DMA,
        ],
        compiler_params=pltpu.CompilerParams(
            use_tc_tiling_on_sc=False, needs_layout_passes=False),
    )
    def gather_kernel(idx_hbm, table_hbm, out_hbm,
                      tab_v, ibufs, obufs, sem_in, sem_out):
        cid = lax.axis_index("c")
        sid = lax.axis_index("s")
        wid = sid * _NC + cid
        row0 = wid * r_per_w

        pltpu.sync_copy(table_hbm, tab_v)
        lane = lax.iota(jnp.int32, 16)
        dq = [(16 * j + lane) // dim for j in range(5)]
        off = [(16 * j + lane) % dim for j in range(5)]

        def expand(ibuf, obuf):
            @plsc.parallel_loop(0, spans, 1, unroll=_UNROLL)
            def _(t):
                for j in range(5):
                    iv = plsc.load_gather(ibuf, [4 * t + dq[j]])
                    val = plsc.load_gather(tab_v, [iv, off[j]])
                    obuf[pl.ds(80 * t + 16 * j, 16)] = val

        for b in range(_NB):
            pltpu.async_copy(
                idx_hbm.at[pl.ds((row0 + b) * _IPR, _IPR)], ibufs[b], sem_in)

        def group(g, carry):
            for b in range(_NB):
                row = row0 + g * _NB + b
                pltpu.make_async_copy(
                    idx_hbm.at[pl.ds(row * _IPR, _IPR)], ibufs[b], sem_in).wait()
                @pl.when(g > 0)
                def _():
                    pltpu.make_async_copy(
                        out_hbm.at[pl.ds(row * odim, odim)], obufs[b],
                        sem_out).wait()
                expand(ibufs[b], obufs[b])
                @pl.when(g < n_groups - 1)
                def _():
                    pltpu.async_copy(
                        idx_hbm.at[pl.ds((row + _NB) * _IPR, _IPR)],
                        ibufs[b], sem_in)
                pltpu.async_copy(
                    obufs[b], out_hbm.at[pl.ds(row * odim, odim)], sem_out)
            return carry

        lax.fori_loop(0, n_groups, group, 0)
        for b in range(_NB):
            pltpu.make_async_copy(
                out_hbm.at[pl.ds(row0 * odim, odim)], obufs[b], sem_out).wait()

    return gather_kernel


def kernel(lengths, table):
    n, s = lengths.shape
    _, dim = table.shape
    m = n * s
    n_rows = m // _IPR
    idx = lengths.reshape(m)
    out = _build(n_rows, dim)(idx, table)
    return out.reshape(n, s, dim)


# speedup vs baseline: 9.7148x; 9.7148x over previous
import functools

import jax
import jax.numpy as jnp
from jax import lax
from jax.experimental import pallas as pl
from jax.experimental.pallas import tpu as pltpu
from jax.experimental.pallas import tpu_sc as plsc

_NC = 2
_NS = 16
_NW = _NC * _NS

_IW = 512
_JB = 8
_NB = 4


@functools.lru_cache(maxsize=None)
def _build(n: int, s: int, dim: int):
    assert n % _IW == 0 and n // _IW == _NW
    n_jb = s // _JB
    assert s % _JB == 0
    mesh = plsc.VectorSubcoreMesh(core_axis_name="c", subcore_axis_name="s")

    @functools.partial(
        pl.kernel,
        mesh=mesh,
        out_type=jax.ShapeDtypeStruct((dim, s, n), jnp.float32),
        scratch_types=[
            pltpu.VMEM((9 * dim,), jnp.float32),
            pltpu.VMEM((_JB, _IW), jnp.int32),
            pltpu.VMEM((_JB, _IW), jnp.int32),
            [pltpu.VMEM((_JB, _IW), jnp.float32)] * _NB,
            pltpu.SemaphoreType.DMA,
            pltpu.SemaphoreType.DMA,
        ],
        compiler_params=pltpu.CompilerParams(
            use_tc_tiling_on_sc=True, needs_layout_passes=False),
    )
    def lut_kernel(idxt_hbm, tabf_hbm, out_hbm,
                   tab_v, ibuf, pbuf, obufs, sem_in, sem_out):
        cid = lax.axis_index("c")
        sid = lax.axis_index("s")
        wid = sid * _NC + cid
        i0 = wid * _IW

        pltpu.sync_copy(tabf_hbm, tab_v)

        def jblock(jt, carry):
            pltpu.async_copy(
                idxt_hbm.at[pl.ds(jt * _JB, _JB), pl.ds(i0, _IW)],
                ibuf, sem_in).wait()

            @plsc.parallel_loop(0, _IW // 16, 1, unroll=4)
            def _(k):
                for jr in range(_JB):
                    pbuf[jr, pl.ds(16 * k, 16)] = (
                        ibuf[jr, pl.ds(16 * k, 16)] * dim)

            for d in range(dim):
                obuf = obufs[d % _NB]
                dst = out_hbm.at[d, pl.ds(jt * _JB, _JB), pl.ds(i0, _IW)]
                if d >= _NB:
                    pltpu.make_async_copy(dst, obuf, sem_out).wait()
                else:
                    @pl.when(jt > 0)
                    def _():
                        pltpu.make_async_copy(dst, obuf, sem_out).wait()

                @plsc.parallel_loop(0, _IW // 16, 1, unroll=4)
                def _(k):
                    for jr in range(_JB):
                        pv = pbuf[jr, pl.ds(16 * k, 16)]
                        obuf[jr, pl.ds(16 * k, 16)] = plsc.load_gather(
                            tab_v, [pv + d])

                pltpu.async_copy(obuf, dst, sem_out)
            return carry

        lax.fori_loop(0, n_jb, jblock, 0)
        for b in range(_NB):
            pltpu.make_async_copy(
                out_hbm.at[0, pl.ds(0, _JB), pl.ds(i0, _IW)],
                obufs[b], sem_out).wait()

    return lut_kernel


def kernel(lengths, table):
    n, s = lengths.shape
    _, dim = table.shape
    idxt = lengths.T
    tabf = table.reshape(9 * dim)
    out_t = _build(n, s, dim)(idxt, tabf)
    return out_t.transpose(2, 1, 0)


# speedup vs baseline: 11.8466x; 1.2194x over previous
import functools

import jax
import jax.numpy as jnp
from jax import lax
from jax.experimental import pallas as pl
from jax.experimental.pallas import tpu as pltpu
from jax.experimental.pallas import tpu_sc as plsc

_NC = 2
_NS = 16
_NW = _NC * _NS

_IW = 512
_JB = 8
_IH = _IW // 2


@functools.lru_cache(maxsize=None)
def _build(n: int, s: int, dim: int):
    assert n % _IW == 0 and n // _IW == _NW
    n_jb = s // _JB
    assert s % _JB == 0
    mesh = plsc.VectorSubcoreMesh(core_axis_name="c", subcore_axis_name="s")

    @functools.partial(
        pl.kernel,
        mesh=mesh,
        out_type=jax.ShapeDtypeStruct((dim, s, n), jnp.float32),
        scratch_types=[
            pltpu.VMEM((9 * dim,), jnp.float32),
            pltpu.VMEM((_JB, _IW), jnp.int32),
            [pltpu.VMEM((_JB, _IH), jnp.float32)] * (2 * dim),
            pltpu.SemaphoreType.DMA,
            pltpu.SemaphoreType.DMA,
        ],
        compiler_params=pltpu.CompilerParams(
            use_tc_tiling_on_sc=True, needs_layout_passes=False),
    )
    def lut_kernel(idxt_hbm, tabtf_hbm, out_hbm,
                   tab_v, ibuf, obufs, sem_in, sem_out):
        cid = lax.axis_index("c")
        sid = lax.axis_index("s")
        wid = sid * _NC + cid
        i0 = wid * _IW

        pltpu.sync_copy(tabtf_hbm, tab_v)

        def dst_of(d, jt, h):
            return out_hbm.at[d, pl.ds(jt * _JB, _JB),
                              pl.ds(i0 + h * _IH, _IH)]

        def jblock(jt, carry):
            pltpu.async_copy(
                idxt_hbm.at[pl.ds(jt * _JB, _JB), pl.ds(i0, _IW)],
                ibuf, sem_in).wait()

            for h in range(2):
                bufs = obufs[h * dim:(h + 1) * dim]
                for d in range(dim):
                    @pl.when(jt > 0)
                    def _():
                        pltpu.make_async_copy(
                            dst_of(d, jt, h), bufs[d], sem_out).wait()

                @plsc.parallel_loop(0, _IH // 16, 1, unroll=2)
                def _(k):
                    for jr in range(_JB):
                        pv = ibuf[jr, pl.ds(h * _IH + 16 * k, 16)]
                        for d in range(dim):
                            bufs[d][jr, pl.ds(16 * k, 16)] = (
                                plsc.load_gather(tab_v, [pv + 9 * d]))

                for d in range(dim):
                    pltpu.async_copy(bufs[d], dst_of(d, jt, h), sem_out)
            return carry

        lax.fori_loop(0, n_jb, jblock, 0)
        for h in range(2):
            for d in range(dim):
                pltpu.make_async_copy(
                    dst_of(d, n_jb - 1, h), obufs[h * dim + d],
                    sem_out).wait()

    return lut_kernel


def kernel(lengths, table):
    n, s = lengths.shape
    _, dim = table.shape
    idxt = lengths.T
    tabtf = table.T.reshape(9 * dim)
    out_t = _build(n, s, dim)(idxt, tabtf)
    return out_t.transpose(2, 1, 0)
